# Initial kernel scaffold; baseline (speedup 1.0000x reference)
import jax
import jax.numpy as jnp
from jax.experimental import pallas as pl


def kernel(input_ids, original_weight, new_weight):
    raise NotImplementedError("write your pallas kernel here")


# speedup vs baseline: 3.2907x; 3.2907x over previous
import functools

import jax
import jax.numpy as jnp
from jax import lax
from jax.experimental import pallas as pl
from jax.experimental.pallas import tpu as pltpu
from jax.experimental.pallas import tpu_sc as plsc

L = 16


@functools.partial(jax.jit, static_argnums=(3, 4))
def _lookup(ids_flat, original_weight, new_weight, n_orig, n_new):
    total = ids_flat.shape[0]
    d = original_weight.shape[1]

    info = plsc.get_sparse_core_info()
    nw = info.num_cores * info.num_subcores
    per_w = total // nw
    chunk = 128
    n_sub = per_w // chunk

    mesh = plsc.VectorSubcoreMesh(core_axis_name="c", subcore_axis_name="s")

    @functools.partial(
        pl.kernel,
        mesh=mesh,
        compiler_params=pltpu.CompilerParams(needs_layout_passes=False),
        out_type=jax.ShapeDtypeStruct((total, d), jnp.float32),
        scratch_types=[
            pltpu.VMEM((n_new, d), jnp.float32),
            pltpu.VMEM((per_w,), jnp.int32),
            pltpu.VMEM((n_sub, chunk), jnp.int32),
            pltpu.VMEM((chunk, d), jnp.float32),
            pltpu.VMEM((chunk, d), jnp.float32),
            pltpu.SemaphoreType.DMA,
            pltpu.SemaphoreType.DMA,
        ],
    )
    def k(ids_hbm, orig_hbm, new_hbm, out_hbm,
          new_v, raw_v, clip_v, rows0, rows1, gsem, osem):
        wid = lax.axis_index("s") * info.num_cores + lax.axis_index("c")
        base = wid * per_w

        pltpu.sync_copy(new_hbm, new_v)
        pltpu.sync_copy(ids_hbm.at[pl.ds(base, per_w)], raw_v)

        lim = jnp.full((L,), n_orig - 1, jnp.int32)
        big = jnp.full((L,), n_orig, jnp.int32)
        sub_has = []
        for sub in range(n_sub):
            h = jnp.int32(0)
            for g in range(chunk // L):
                v = raw_v[pl.ds(sub * chunk + g * L, L)]
                clip_v[sub, pl.ds(g * L, L)] = jnp.minimum(v, lim)
                cnt = plsc.all_reduce_population_count(v >= big)
                h = h + cnt[0]
            sub_has.append(h > 0)

        rows = [rows0, rows1]

        def fire_gather(sub):
            return pltpu.async_copy(
                orig_hbm.at[clip_v.at[sub]], rows[sub % 2], gsem)

        def patch(sub, buf):
            @pl.when(sub_has[sub])
            def _():
                for g in range(chunk // L):
                    v = raw_v[pl.ds(sub * chunk + g * L, L)]
                    mask = v >= jnp.full((L,), n_orig, jnp.int32)

                    @pl.when(plsc.all_reduce_population_count(mask)[0] > 0)
                    def _():
                        nidx = jnp.clip(v - n_orig, 0, n_new - 1)
                        row_ids = jnp.arange(g * L, (g + 1) * L, dtype=jnp.int32)

                        def col_body(cb, carry):
                            cvec = jnp.full((L,), cb, jnp.int32)
                            vals = plsc.load_gather(new_v, [nidx, cvec], mask=mask)
                            plsc.store_scatter(buf, [row_ids, cvec], vals,
                                               mask=mask)
                            return carry

                        lax.fori_loop(0, d, col_body, 0)

        cps = [None] * n_sub
        out_cp = [None] * n_sub
        cps[0] = fire_gather(0)
        for sub in range(n_sub):
            cps[sub].wait()
            if sub + 1 < n_sub:
                if sub >= 1:
                    out_cp[sub - 1].wait()
                cps[sub + 1] = fire_gather(sub + 1)
            patch(sub, rows[sub % 2])
            out_cp[sub] = pltpu.async_copy(
                rows[sub % 2],
                out_hbm.at[pl.ds(base + sub * chunk, chunk)], osem)
        out_cp[n_sub - 2].wait()
        out_cp[n_sub - 1].wait()

    return k(ids_flat, original_weight, new_weight)


def kernel(input_ids, original_weight, new_weight):
    b, s = input_ids.shape
    n_orig, d = original_weight.shape
    n_new = new_weight.shape[0]
    ids_flat = input_ids.reshape(-1).astype(jnp.int32)
    out = _lookup(ids_flat, original_weight, new_weight, n_orig, n_new)
    return out.reshape(b, s, d)


# speedup vs baseline: 3.6406x; 1.1064x over previous
import functools

import jax
import jax.numpy as jnp
from jax import lax
from jax.experimental import pallas as pl
from jax.experimental.pallas import tpu as pltpu
from jax.experimental.pallas import tpu_sc as plsc

L = 16
NBUF = 6
DEPTH = 4


@functools.partial(jax.jit, static_argnums=(3, 4))
def _lookup(ids_flat, original_weight, new_weight, n_orig, n_new):
    total = ids_flat.shape[0]
    d = original_weight.shape[1]

    info = plsc.get_sparse_core_info()
    nw = info.num_cores * info.num_subcores
    per_w = total // nw
    chunk = 128
    n_sub = per_w // chunk

    mesh = plsc.VectorSubcoreMesh(core_axis_name="c", subcore_axis_name="s")

    @functools.partial(
        pl.kernel,
        mesh=mesh,
        compiler_params=pltpu.CompilerParams(needs_layout_passes=False),
        out_type=jax.ShapeDtypeStruct((total, d), jnp.float32),
        scratch_types=[
            pltpu.VMEM((n_new, d), jnp.float32),
            pltpu.VMEM((per_w,), jnp.int32),
            pltpu.VMEM((n_sub, chunk), jnp.int32),
            *([pltpu.VMEM((chunk, d), jnp.float32)] * NBUF),
            pltpu.SemaphoreType.DMA,
            pltpu.SemaphoreType.DMA,
            pltpu.SemaphoreType.DMA,
        ],
    )
    def k(ids_hbm, orig_hbm, new_hbm, out_hbm,
          new_v, raw_v, clip_v, *rows_and_sems):
        rows = rows_and_sems[:NBUF]
        gsem, osem, nsem = rows_and_sems[NBUF:]
        wid = lax.axis_index("s") * info.num_cores + lax.axis_index("c")
        base = wid * per_w

        cp_new = pltpu.async_copy(new_hbm, new_v, nsem)
        pltpu.sync_copy(ids_hbm.at[pl.ds(base, per_w)], raw_v)

        lim = jnp.full((L,), n_orig - 1, jnp.int32)
        big = jnp.full((L,), n_orig, jnp.int32)

        def clip_sub(sub):
            h = jnp.int32(0)
            for g in range(chunk // L):
                v = raw_v[pl.ds(sub * chunk + g * L, L)]
                clip_v[sub, pl.ds(g * L, L)] = jnp.minimum(v, lim)
                h = h + plsc.all_reduce_population_count(v >= big)[0]
            return h > 0

        def fire_gather(sub):
            return pltpu.async_copy(
                orig_hbm.at[clip_v.at[sub]], rows[sub % NBUF], gsem)

        def fire_out(sub):
            return pltpu.async_copy(
                rows[sub % NBUF],
                out_hbm.at[pl.ds(base + sub * chunk, chunk)], osem)

        def patch(sub, has, buf):
            @pl.when(has)
            def _():
                for g in range(chunk // L):
                    v = raw_v[pl.ds(sub * chunk + g * L, L)]
                    mask = v >= big

                    @pl.when(plsc.all_reduce_population_count(mask)[0] > 0)
                    def _():
                        nidx = jnp.clip(v - n_orig, 0, n_new - 1)
                        row_ids = jnp.arange(g * L, (g + 1) * L, dtype=jnp.int32)

                        def col_body(cb, carry):
                            cvec = jnp.full((L,), cb, jnp.int32)
                            vals = plsc.load_gather(new_v, [nidx, cvec], mask=mask)
                            plsc.store_scatter(buf, [row_ids, cvec], vals,
                                               mask=mask)
                            return carry

                        lax.fori_loop(0, d, col_body, 0)

        sub_has = [None] * n_sub
        gcp = [None] * n_sub
        ocp = [None] * n_sub
        head = min(DEPTH, n_sub)
        for sub in range(head):
            sub_has[sub] = clip_sub(sub)
            gcp[sub] = fire_gather(sub)
        cp_new.wait()
        for sub in range(head, n_sub):
            sub_has[sub] = clip_sub(sub)

        for sub in range(n_sub):
            gcp[sub].wait()
            patch(sub, sub_has[sub], rows[sub % NBUF])
            ocp[sub] = fire_out(sub)
            nxt = sub + DEPTH
            if nxt < n_sub:
                if nxt >= NBUF:
                    ocp[nxt - NBUF].wait()
                gcp[nxt] = fire_gather(nxt)
        waited = set()
        for sub in range(n_sub):
            nxt = sub + DEPTH
            if nxt < n_sub and nxt >= NBUF:
                waited.add(nxt - NBUF)
        for j in range(n_sub):
            if j not in waited:
                ocp[j].wait()

    return k(ids_flat, original_weight, new_weight)


def kernel(input_ids, original_weight, new_weight):
    b, s = input_ids.shape
    n_orig, d = original_weight.shape
    n_new = new_weight.shape[0]
    ids_flat = input_ids.reshape(-1).astype(jnp.int32)
    out = _lookup(ids_flat, original_weight, new_weight, n_orig, n_new)
    return out.reshape(b, s, d)
